# baseline (speedup 1.0000x reference)
import functools

import jax
import jax.numpy as jnp
from jax import lax
from jax.experimental import pallas as pl
from jax.experimental.pallas import tpu as pltpu
from jax.experimental.pallas import tpu_sc as plsc

BATCH = 64
SEQ = 512
EMBD = 512
NW = 32
TOK_PER_W = (BATCH * SEQ) // NW
CHUNK = 64
NCHUNK = TOK_PER_W // CHUNK
LANES = 16
VECS = EMBD // LANES


def _emb_body(x_hbm, tok_hbm, pos_hbm, out_hbm, idx_v, rows_v, pos_v):
    wid = lax.axis_index("s") * 2 + lax.axis_index("c")
    base = wid * TOK_PER_W
    for c in range(NCHUNK):
        row0 = base + c * CHUNK
        p0 = (c % (SEQ // CHUNK)) * CHUNK
        pltpu.sync_copy(x_hbm.at[pl.ds(row0, CHUNK)], idx_v)
        pltpu.sync_copy(pos_hbm.at[pl.ds(p0, CHUNK)], pos_v)
        pltpu.sync_copy(tok_hbm.at[idx_v], rows_v)

        def add_row(r, carry):
            for j in range(VECS):
                sl = pl.ds(j * LANES, LANES)
                rows_v[r, sl] = rows_v[r, sl] + pos_v[r, sl]
            return carry

        lax.fori_loop(0, CHUNK, add_row, 0)
        pltpu.sync_copy(rows_v, out_hbm.at[pl.ds(row0, CHUNK)])


def kernel(x, token_table, position_table):
    xf = x.reshape(-1).astype(jnp.int32)
    mesh = plsc.VectorSubcoreMesh(core_axis_name="c", subcore_axis_name="s")
    f = functools.partial(
        pl.kernel,
        mesh=mesh,
        out_type=jax.ShapeDtypeStruct((BATCH * SEQ, EMBD), jnp.float32),
        scratch_types=[
            pltpu.VMEM((CHUNK,), jnp.int32),
            pltpu.VMEM((CHUNK, EMBD), jnp.float32),
            pltpu.VMEM((CHUNK, EMBD), jnp.float32),
        ],
    )(_emb_body)
    out = f(xf, token_table, position_table)
    return out.reshape(BATCH, SEQ, EMBD)
